# baseline (speedup 1.0000x reference)
import functools

import jax
import jax.numpy as jnp
from jax import lax
from jax.experimental import pallas as pl
from jax.experimental.pallas import tpu as pltpu
from jax.experimental.pallas import tpu_sc as plsc

D = 1024
FF = 1024
E = 64
N = 4096
TILE = 64
ALIGN = 8
NPAD = N + E * (ALIGN - 1) + TILE

_NC = 2
_NS = 16
_NW = _NC * _NS
_CHUNK = 64
_ROWS_PER_W = N // _NW


def _router_body(x_ref, wr_ref, pos_ref, w_ref, sc_ref):
    xf = x_ref[...]
    wr = wr_ref[...]
    logits = lax.dot_general(
        xf, wr, (((1,), (1,)), ((), ())), preferred_element_type=jnp.float32
    )
    rowmax = jnp.max(logits, axis=1, keepdims=True)
    denom = jnp.sum(jnp.exp(logits - rowmax), axis=1, keepdims=True)
    p = 1.0 / denom
    w = p / (p + 1e-8)

    iota_e = lax.broadcasted_iota(jnp.int32, (N, E), 1).astype(jnp.float32)
    is_max = logits == rowmax
    e_tok = jnp.min(jnp.where(is_max, iota_e, float(E)), axis=1, keepdims=True)
    onehot = (iota_e == e_tok).astype(jnp.float32)

    counts = jnp.sum(onehot, axis=0, keepdims=True)
    padded = jnp.ceil(counts / ALIGN) * ALIGN
    ia = lax.broadcasted_iota(jnp.int32, (E, E), 0)
    ib = lax.broadcasted_iota(jnp.int32, (E, E), 1)
    strict_lt = (ia < ib).astype(jnp.float32)
    offsets = lax.dot_general(
        padded, strict_lt, (((1,), (0,)), ((), ())),
        preferred_element_type=jnp.float32,
    )

    blk = 512
    nb = N // blk
    blkcnt = jnp.concatenate(
        [jnp.sum(onehot[b * blk:(b + 1) * blk, :], axis=0, keepdims=True)
         for b in range(nb)], axis=0)
    sa = lax.broadcasted_iota(jnp.int32, (nb, nb), 0)
    sb = lax.broadcasted_iota(jnp.int32, (nb, nb), 1)
    strict_blk = (sb < sa).astype(jnp.float32)
    base = lax.dot_general(
        strict_blk, blkcnt, (((1,), (0,)), ((), ())),
        preferred_element_type=jnp.float32,
    )
    jj = lax.broadcasted_iota(jnp.int32, (blk, blk), 1)
    ii = lax.broadcasted_iota(jnp.int32, (blk, blk), 0)
    strict_in = (jj < ii).astype(jnp.float32)
    for b in range(nb):
        r0 = b * blk
        mb = onehot[r0:r0 + blk, :]
        same = lax.dot_general(
            mb, mb, (((1,), (1,)), ((), ())),
            preferred_element_type=jnp.float32,
        )
        rank = jnp.sum(same * strict_in, axis=1, keepdims=True)
        off_b = offsets + base[b:b + 1, :]
        tok_off = jnp.sum(mb * off_b, axis=1, keepdims=True)
        pos_ref[r0:r0 + blk, :] = (tok_off + rank).astype(jnp.int32)

    w_ref[...] = jnp.broadcast_to(w, (N, 128))
    table = jnp.concatenate(
        [offsets.astype(jnp.int32), counts.astype(jnp.int32),
         jnp.zeros((6, E), jnp.int32)], axis=0)
    sc_ref[...] = table


_router_call = pl.pallas_call(
    _router_body,
    out_shape=(
        jax.ShapeDtypeStruct((N, 1), jnp.int32),
        jax.ShapeDtypeStruct((N, 128), jnp.float32),
        jax.ShapeDtypeStruct((8, E), jnp.int32),
    ),
)


def _expert_body(sc_ref, xs_ref, ws_ref, wg_ref, wu_ref, wd_ref, out_ref):
    e = pl.program_id(0)
    start = pl.multiple_of(sc_ref[e], ALIGN)
    count = sc_ref[E + e]
    ntiles = (count + TILE - 1) // TILE

    def tile_body(t, carry):
        r = start + t * TILE
        xb = xs_ref[pl.ds(r, TILE), :]
        g = lax.dot_general(
            xb, wg_ref[0], (((1,), (1,)), ((), ())),
            preferred_element_type=jnp.float32)
        u = lax.dot_general(
            xb, wu_ref[0], (((1,), (1,)), ((), ())),
            preferred_element_type=jnp.float32)
        h = g * jax.nn.sigmoid(g) * u
        o = lax.dot_general(
            h, wd_ref[0], (((1,), (1,)), ((), ())),
            preferred_element_type=jnp.float32)
        o = o * ws_ref[pl.ds(r, TILE), 0:1]
        out_ref[pl.ds(r, TILE), :] = o
        return carry

    lax.fori_loop(0, ntiles, tile_body, 0)


_expert_call = pl.pallas_call(
    _expert_body,
    grid_spec=pltpu.PrefetchScalarGridSpec(
        num_scalar_prefetch=1,
        grid=(E,),
        in_specs=[
            pl.BlockSpec((NPAD, D), lambda e, sc: (0, 0)),
            pl.BlockSpec((NPAD, 128), lambda e, sc: (0, 0)),
            pl.BlockSpec((1, FF, D), lambda e, sc: (e, 0, 0)),
            pl.BlockSpec((1, FF, D), lambda e, sc: (e, 0, 0)),
            pl.BlockSpec((1, D, FF), lambda e, sc: (e, 0, 0)),
        ],
        out_specs=pl.BlockSpec((NPAD, D), lambda e, sc: (0, 0)),
    ),
    out_shape=jax.ShapeDtypeStruct((NPAD, D), jnp.float32),
    compiler_params=pltpu.CompilerParams(vmem_limit_bytes=100 * 1024 * 1024),
)

@functools.cache
def _build_sc_kernels():
    mesh = plsc.VectorSubcoreMesh(core_axis_name="c", subcore_axis_name="s")

    @functools.partial(
        pl.kernel,
        out_type=(
            jax.ShapeDtypeStruct((NPAD, D), jnp.float32),
            jax.ShapeDtypeStruct((NPAD, 128), jnp.float32),
        ),
        mesh=mesh,
        scratch_types=(
            pltpu.VMEM((_CHUNK,), jnp.int32),
            pltpu.VMEM((_CHUNK, D), jnp.float32),
            pltpu.VMEM((_CHUNK, 128), jnp.float32),
            pltpu.SemaphoreType.DMA,
        ),
    )
    def _scatter_tokens(x_hbm, w_hbm, pos_hbm, xs_hbm, ws_hbm, idx_v, row_v,
                        wrow_v, sem):
        wid = lax.axis_index("s") * _NC + lax.axis_index("c")
        base = wid * _ROWS_PER_W
        for it in range(_ROWS_PER_W // _CHUNK):
            b = base + it * _CHUNK
            pltpu.sync_copy(pos_hbm.at[pl.ds(b, _CHUNK)], idx_v)
            pltpu.sync_copy(x_hbm.at[pl.ds(b, _CHUNK)], row_v)
            pltpu.sync_copy(w_hbm.at[pl.ds(b, _CHUNK)], wrow_v)
            pltpu.async_copy(row_v, xs_hbm.at[idx_v], sem).wait()
            pltpu.async_copy(wrow_v, ws_hbm.at[idx_v], sem).wait()

    @functools.partial(
        pl.kernel,
        out_type=jax.ShapeDtypeStruct((N, D), jnp.float32),
        mesh=mesh,
        scratch_types=(
            pltpu.VMEM((_CHUNK,), jnp.int32),
            pltpu.VMEM((_CHUNK, D), jnp.float32),
            pltpu.SemaphoreType.DMA,
        ),
    )
    def _gather_out(outs_hbm, pos_hbm, out_hbm, idx_v, row_v, sem):
        wid = lax.axis_index("s") * _NC + lax.axis_index("c")
        base = wid * _ROWS_PER_W
        for it in range(_ROWS_PER_W // _CHUNK):
            b = base + it * _CHUNK
            pltpu.sync_copy(pos_hbm.at[pl.ds(b, _CHUNK)], idx_v)
            pltpu.async_copy(outs_hbm.at[idx_v], row_v, sem).wait()
            pltpu.sync_copy(row_v, out_hbm.at[pl.ds(b, _CHUNK)])

    return _scatter_tokens, _gather_out


def kernel(x, W_router, W_gate, W_up, W_down):
    Bx, Tx, Dx = x.shape
    xf = x.reshape(Bx * Tx, Dx)
    pos2d, w16, table = _router_call(xf, W_router)
    pos = pos2d.reshape(-1)
    scal = jnp.concatenate([table[0], table[1]])
    _scatter_tokens, _gather_out = _build_sc_kernels()
    xs, ws = _scatter_tokens(xf, w16, pos)
    outs = _expert_call(scal, xs, ws, W_gate, W_up, W_down)
    out = _gather_out(outs, pos)
    return out.reshape(Bx, Tx, Dx)


# speedup vs baseline: 1.1220x; 1.1220x over previous
import functools

import jax
import jax.numpy as jnp
from jax import lax
from jax.experimental import pallas as pl
from jax.experimental.pallas import tpu as pltpu
from jax.experimental.pallas import tpu_sc as plsc

D = 1024
FF = 1024
E = 64
N = 4096
TILE = 128
ALIGN = 8
NPAD = N + E * (ALIGN - 1) + TILE

_NC = 2
_NS = 16
_NW = _NC * _NS
_CHUNK = 64
_ROWS_PER_W = N // _NW


def _router_body(x_ref, wr_ref, pos_ref, sc_ref):
    xf = x_ref[...]
    wr = wr_ref[...]
    logits = lax.dot_general(
        xf, wr, (((1,), (1,)), ((), ())), preferred_element_type=jnp.float32
    )
    rowmax = jnp.max(logits, axis=1, keepdims=True)
    iota_e = lax.broadcasted_iota(jnp.int32, (N, E), 1).astype(jnp.float32)
    is_max = logits == rowmax
    e_tok = jnp.min(jnp.where(is_max, iota_e, float(E)), axis=1, keepdims=True)
    onehot = (iota_e == e_tok).astype(jnp.float32)

    counts = jnp.sum(onehot, axis=0, keepdims=True)
    padded = jnp.ceil(counts / ALIGN) * ALIGN
    ia = lax.broadcasted_iota(jnp.int32, (E, E), 0)
    ib = lax.broadcasted_iota(jnp.int32, (E, E), 1)
    strict_lt = (ia < ib).astype(jnp.float32)
    offsets = lax.dot_general(
        padded, strict_lt, (((1,), (0,)), ((), ())),
        preferred_element_type=jnp.float32,
    )

    blk = 512
    nb = N // blk
    blkcnt = jnp.concatenate(
        [jnp.sum(onehot[b * blk:(b + 1) * blk, :], axis=0, keepdims=True)
         for b in range(nb)], axis=0)
    sa = lax.broadcasted_iota(jnp.int32, (nb, nb), 0)
    sb = lax.broadcasted_iota(jnp.int32, (nb, nb), 1)
    strict_blk = (sb < sa).astype(jnp.float32)
    base = lax.dot_general(
        strict_blk, blkcnt, (((1,), (0,)), ((), ())),
        preferred_element_type=jnp.float32,
    )
    jj = lax.broadcasted_iota(jnp.int32, (blk, blk), 1)
    ii = lax.broadcasted_iota(jnp.int32, (blk, blk), 0)
    strict_in = (jj < ii).astype(jnp.float32)
    for b in range(nb):
        r0 = b * blk
        mb = onehot[r0:r0 + blk, :]
        same = lax.dot_general(
            mb, mb, (((1,), (1,)), ((), ())),
            preferred_element_type=jnp.float32,
        )
        rank = jnp.sum(same * strict_in, axis=1, keepdims=True)
        off_b = offsets + base[b:b + 1, :]
        tok_off = jnp.sum(mb * off_b, axis=1, keepdims=True)
        pos_ref[r0:r0 + blk, :] = (tok_off + rank).astype(jnp.int32)

    table = jnp.concatenate(
        [offsets.astype(jnp.int32), counts.astype(jnp.int32),
         jnp.zeros((6, E), jnp.int32)], axis=0)
    sc_ref[...] = table


_router_call = pl.pallas_call(
    _router_body,
    out_shape=(
        jax.ShapeDtypeStruct((N, 1), jnp.int32),
        jax.ShapeDtypeStruct((8, E), jnp.int32),
    ),
)


def _expert_body(sc_ref, xs_ref, wg_ref, wu_ref, wd_ref, out_ref):
    e = pl.program_id(0)
    start = pl.multiple_of(sc_ref[e], ALIGN)
    count = sc_ref[E + e]
    ntiles = (count + TILE - 1) // TILE

    def tile_body(t, carry):
        r = start + t * TILE
        xb = xs_ref[pl.ds(r, TILE), :]
        g = lax.dot_general(
            xb, wg_ref[0], (((1,), (1,)), ((), ())),
            preferred_element_type=jnp.float32)
        u = lax.dot_general(
            xb, wu_ref[0], (((1,), (1,)), ((), ())),
            preferred_element_type=jnp.float32)
        h = g * jax.nn.sigmoid(g) * u
        o = lax.dot_general(
            h, wd_ref[0], (((1,), (1,)), ((), ())),
            preferred_element_type=jnp.float32)
        out_ref[pl.ds(r, TILE), :] = o
        return carry

    lax.fori_loop(0, ntiles, tile_body, 0)


_expert_call = pl.pallas_call(
    _expert_body,
    grid_spec=pltpu.PrefetchScalarGridSpec(
        num_scalar_prefetch=1,
        grid=(E,),
        in_specs=[
            pl.BlockSpec((NPAD, D), lambda e, sc: (0, 0)),
            pl.BlockSpec((1, FF, D), lambda e, sc: (e, 0, 0)),
            pl.BlockSpec((1, FF, D), lambda e, sc: (e, 0, 0)),
            pl.BlockSpec((1, D, FF), lambda e, sc: (e, 0, 0)),
        ],
        out_specs=pl.BlockSpec((NPAD, D), lambda e, sc: (0, 0)),
    ),
    out_shape=jax.ShapeDtypeStruct((NPAD, D), jnp.float32),
    compiler_params=pltpu.CompilerParams(vmem_limit_bytes=100 * 1024 * 1024),
)

@functools.cache
def _build_sc_kernels():
    mesh = plsc.VectorSubcoreMesh(core_axis_name="c", subcore_axis_name="s")

    @functools.partial(
        pl.kernel,
        out_type=jax.ShapeDtypeStruct((NPAD, D), jnp.float32),
        mesh=mesh,
        scratch_types=(
            pltpu.VMEM((_CHUNK,), jnp.int32),
            pltpu.VMEM((_CHUNK, D), jnp.float32),
            pltpu.SemaphoreType.DMA,
        ),
    )
    def _scatter_tokens(x_hbm, pos_hbm, xs_hbm, idx_v, row_v, sem):
        wid = lax.axis_index("s") * _NC + lax.axis_index("c")
        base = wid * _ROWS_PER_W
        for it in range(_ROWS_PER_W // _CHUNK):
            b = base + it * _CHUNK
            pltpu.sync_copy(pos_hbm.at[pl.ds(b, _CHUNK)], idx_v)
            pltpu.sync_copy(x_hbm.at[pl.ds(b, _CHUNK)], row_v)
            pltpu.async_copy(row_v, xs_hbm.at[idx_v], sem).wait()

    @functools.partial(
        pl.kernel,
        out_type=jax.ShapeDtypeStruct((N, D), jnp.float32),
        mesh=mesh,
        scratch_types=(
            pltpu.VMEM((_CHUNK,), jnp.int32),
            pltpu.VMEM((_CHUNK, D), jnp.float32),
            pltpu.SemaphoreType.DMA,
        ),
    )
    def _gather_out(outs_hbm, pos_hbm, out_hbm, idx_v, row_v, sem):
        wid = lax.axis_index("s") * _NC + lax.axis_index("c")
        base = wid * _ROWS_PER_W
        for it in range(_ROWS_PER_W // _CHUNK):
            b = base + it * _CHUNK
            pltpu.sync_copy(pos_hbm.at[pl.ds(b, _CHUNK)], idx_v)
            pltpu.async_copy(outs_hbm.at[idx_v], row_v, sem).wait()
            pltpu.sync_copy(row_v, out_hbm.at[pl.ds(b, _CHUNK)])

    return _scatter_tokens, _gather_out


def kernel(x, W_router, W_gate, W_up, W_down):
    Bx, Tx, Dx = x.shape
    xf = x.reshape(Bx * Tx, Dx)
    pos2d, table = _router_call(xf, W_router)
    pos = pos2d.reshape(-1)
    scal = jnp.concatenate([table[0], table[1]])
    _scatter_tokens, _gather_out = _build_sc_kernels()
    xs = _scatter_tokens(xf, pos)
    outs = _expert_call(scal, xs, W_gate, W_up, W_down)
    out = _gather_out(outs, pos)
    return out.reshape(Bx, Tx, Dx)
